# baseline (speedup 1.0000x reference)
import functools
import jax
import jax.numpy as jnp
from jax import lax
from jax.experimental import pallas as pl
from jax.experimental.pallas import tpu as pltpu
from jax.experimental.pallas import tpu_sc as plsc

_B, _N, _MO, _D = 128, 32, 48, 64
_R = _N * _B
_NB = 4
_L = 16
_G = _B // _L


def _sc_cnt_body(cats_hbm, subs_hbm, mask_hbm, cl_hbm, cr_hbm, cnt_hbm,
                 cats_v, subs_v, mask_v, cl_v, cr_v, vc_v,
                 vc_pub, cnt_o, vc_sh):
    c = lax.axis_index("c")
    s = lax.axis_index("s")

    pltpu.sync_copy(cats_hbm, cats_v)
    pltpu.sync_copy(subs_hbm, subs_v)
    pltpu.sync_copy(mask_hbm, mask_v)
    pltpu.sync_copy(cl_hbm, cl_v)
    pltpu.sync_copy(cr_hbm, cr_v)

    lane = lax.iota(jnp.int32, _L)

    for r in range(2):
        row = 2 * s + r
        for g in range(_G):
            sl = pl.ds(row * _B + g * _L, _L)
            b_idx = lane + g * _L
            cats_g = cats_v[sl]
            subs_g = subs_v[sl]
            mask_g = mask_v[sl]
            cl_g = jnp.clip(cl_v[sl], 0, _N - 1)
            flat_l = cl_g * _B + b_idx
            cats_l = plsc.load_gather(cats_v, [flat_l])
            subs_l = plsc.load_gather(subs_v, [flat_l])
            mask_l = plsc.load_gather(mask_v, [flat_l])
            ecat = jnp.where(mask_g != 0, cats_g, 3)
            ecat_l = jnp.where(mask_l != 0, cats_l, 3)
            vpm = jnp.where(ecat == 0, subs_g + 1, subs_l + 1)
            cpm = jnp.where(ecat == 0, 1,
                            jnp.where((ecat == 1) & (ecat_l == 0), subs_g + 2, 0))
            vc_pub[pl.ds(r * _B + g * _L, _L)] = vpm + 8 * cpm

    pltpu.sync_copy(vc_pub, vc_sh.at[pl.ds(2 * s * _B, 2 * _B)])
    plsc.subcore_barrier()
    pltpu.sync_copy(vc_sh, vc_v)

    w = 2 * s + c
    for g in range(_G):
        sl = pl.ds(w * _B + g * _L, _L)
        b_idx = lane + g * _L
        cats_g = cats_v[sl]
        subs_g = subs_v[sl]
        mask_g = mask_v[sl]
        cl_g = jnp.clip(cl_v[sl], 0, _N - 1)
        cr_g = jnp.clip(cr_v[sl], 0, _N - 1)
        vc_g = vc_v[sl]
        ecat = jnp.where(mask_g != 0, cats_g, 3)
        is_after = subs_g == 1
        i_first = jnp.where(is_after, cr_g, cl_g)
        i_second = jnp.where(is_after, cl_g, cr_g)
        pk_f = plsc.load_gather(vc_v, [i_first * _B + b_idx])
        pk_s = plsc.load_gather(vc_v, [i_second * _B + b_idx])
        is_comb = ecat == 2
        c_a = jnp.where(is_comb, pk_f >> 3, vc_g >> 3)
        c_b = jnp.where(is_comb, pk_s >> 3, 0)
        cnt_o[pl.ds(g * _L, _L)] = (c_a + c_b).astype(jnp.float32)

    pltpu.sync_copy(cnt_o, cnt_hbm.at[pl.ds(w * _B, _B)])


_sc_cnt = functools.partial(
    pl.kernel,
    out_type=jax.ShapeDtypeStruct((_R,), jnp.float32),
    mesh=plsc.VectorSubcoreMesh(core_axis_name="c", subcore_axis_name="s"),
    compiler_params=pltpu.CompilerParams(needs_layout_passes=False),
    scratch_types=[pltpu.VMEM((_R,), jnp.int32)] * 6
    + [pltpu.VMEM((2 * _B,), jnp.int32),
       pltpu.VMEM((_B,), jnp.float32),
       pltpu.VMEM_SHARED((_R,), jnp.int32)],
)(_sc_cnt_body)


def _loop_gather(x, idx, rows):
    acc = jnp.zeros((rows, _B), x.dtype)
    for j in range(_N):
        acc = jnp.where(idx == j, x[j:j + 1, :], acc)
    return acc


def _tc_body(cats_ref, subs_ref, mask_ref, cl_ref, cr_ref,
             catsb_ref, subsb_ref, maskb_ref, clb_ref, crb_ref,
             e1_ref, e2_ref, out_ref):
    cats = cats_ref[...]
    subs = subs_ref[...]
    msk = mask_ref[...]
    cl = jnp.clip(cl_ref[...], 0, _N - 1)

    ecat = jnp.where(msk != 0, cats, 3)
    pk_l = _loop_gather(ecat + 4 * subs, cl, _N)
    vpm = jnp.where(ecat == 0, subs + 1, (pk_l >> 2) + 1)
    cpm = jnp.where(ecat == 0, 1,
                    jnp.where((ecat == 1) & ((pk_l & 3) == 0), subs + 2, 0))
    vc = vpm + 8 * cpm

    cats_b = catsb_ref[0]
    subs_b = subsb_ref[0]
    mask_b = maskb_ref[0]
    cl_b = jnp.clip(clb_ref[0], 0, _N - 1)
    cr_b = jnp.clip(crb_ref[0], 0, _N - 1)
    ecat_b = jnp.where(mask_b != 0, cats_b, 3)
    pk_lb = _loop_gather(ecat + 4 * subs, cl_b, _NB)
    vpm_b = jnp.where(ecat_b == 0, subs_b + 1, (pk_lb >> 2) + 1)
    cpm_b = jnp.where(ecat_b == 0, 1,
                      jnp.where((ecat_b == 1) & ((pk_lb & 3) == 0),
                                subs_b + 2, 0))
    is_after = subs_b == 1
    i_first = jnp.where(is_after, cr_b, cl_b)
    i_second = jnp.where(is_after, cl_b, cr_b)
    pk_f = _loop_gather(vc, i_first, _NB)
    pk_s = _loop_gather(vc, i_second, _NB)
    is_comb = ecat_b == 2
    c_a = jnp.where(is_comb, pk_f >> 3, cpm_b)
    v_a = jnp.where(is_comb, pk_f & 7, vpm_b)
    c_b = jnp.where(is_comb, pk_s >> 3, 0)
    v_b = pk_s & 7

    e1 = e1_ref[...]
    e2 = e2_ref[...]
    zero = jnp.zeros((1, 1, 1), jnp.float32)
    p3 = lax.broadcasted_iota(jnp.int32, (_MO, 1, 1), 0)
    cab = c_a + c_b
    for k in range(_NB):
        ca3 = lax.broadcast_in_dim(c_a[k:k + 1, :], (1, 1, _B), (1, 2))
        cab3 = lax.broadcast_in_dim(cab[k:k + 1, :], (1, 1, _B), (1, 2))
        va3 = lax.broadcast_in_dim(v_a[k:k + 1, :], (1, 1, _B), (1, 2))
        vb3 = lax.broadcast_in_dim(v_b[k:k + 1, :], (1, 1, _B), (1, 2))
        ea = jnp.where(va3 == 1, e1, jnp.where(va3 == 2, e2, zero))
        eb = jnp.where(vb3 == 1, e1, jnp.where(vb3 == 2, e2, zero))
        in_a = p3 < ca3
        in_ab = p3 < cab3
        out_ref[k] = jnp.where(in_a, ea, jnp.where(in_ab, eb, zero))


def kernel(node_cats, node_subs, node_mask, child_left, child_right, action_embed):
    mask_i = node_mask.astype(jnp.int32)
    ct, st, mt = node_cats.T, node_subs.T, mask_i.T
    lt, rt = child_left.T, child_right.T

    cnt = _sc_cnt(ct.reshape(_R), st.reshape(_R), mt.reshape(_R),
                  lt.reshape(_R), rt.reshape(_R))

    g = _N // _NB
    full_spec = pl.BlockSpec((_N, _B), lambda i: (0, 0))
    blk_spec = pl.BlockSpec((1, _NB, _B), lambda i: (i, 0, 0))
    evec_spec = pl.BlockSpec((1, _D, 1), lambda i: (0, 0, 0))
    out = pl.pallas_call(
        _tc_body,
        grid=(g,),
        in_specs=[full_spec] * 5 + [blk_spec] * 5 + [evec_spec] * 2,
        out_specs=pl.BlockSpec((_NB, _MO, _D, _B), lambda i: (i, 0, 0, 0)),
        out_shape=jax.ShapeDtypeStruct((_N, _MO, _D, _B), jnp.float32),
    )(ct, st, mt, lt, rt,
      ct.reshape(g, _NB, _B), st.reshape(g, _NB, _B), mt.reshape(g, _NB, _B),
      lt.reshape(g, _NB, _B), rt.reshape(g, _NB, _B),
      action_embed[1].reshape(1, _D, 1), action_embed[2].reshape(1, _D, 1))

    return jnp.transpose(out, (3, 0, 1, 2)), cnt.reshape(_N, _B).T


# speedup vs baseline: 2.0005x; 2.0005x over previous
import jax
import jax.numpy as jnp
from jax import lax
from jax.experimental import pallas as pl

_B, _N, _MO, _D = 128, 32, 48, 64
_NB = 4


def _loop_gather(x, idx, rows):
    acc = jnp.zeros((rows, _B), x.dtype)
    for j in range(_N):
        acc = jnp.where(idx == j, x[j:j + 1, :], acc)
    return acc


def _body(cats_ref, subs_ref, mask_ref, cl_ref, cr_ref,
          catsb_ref, subsb_ref, maskb_ref, clb_ref, crb_ref,
          e1_ref, e2_ref, out_ref, cnt_ref):
    cats = cats_ref[...]
    subs = subs_ref[...]
    msk = mask_ref[...]
    cl = jnp.clip(cl_ref[...], 0, _N - 1)

    ecat = jnp.where(msk != 0, cats, 3)
    pk_l = _loop_gather(ecat + 4 * subs, cl, _N)
    vpm = jnp.where(ecat == 0, subs + 1, (pk_l >> 2) + 1)
    cpm = jnp.where(ecat == 0, 1,
                    jnp.where((ecat == 1) & ((pk_l & 3) == 0), subs + 2, 0))
    vc = vpm + 8 * cpm

    cats_b = catsb_ref[0]
    subs_b = subsb_ref[0]
    mask_b = maskb_ref[0]
    cl_b = jnp.clip(clb_ref[0], 0, _N - 1)
    cr_b = jnp.clip(crb_ref[0], 0, _N - 1)
    ecat_b = jnp.where(mask_b != 0, cats_b, 3)
    pk_lb = _loop_gather(ecat + 4 * subs, cl_b, _NB)
    vpm_b = jnp.where(ecat_b == 0, subs_b + 1, (pk_lb >> 2) + 1)
    cpm_b = jnp.where(ecat_b == 0, 1,
                      jnp.where((ecat_b == 1) & ((pk_lb & 3) == 0),
                                subs_b + 2, 0))
    is_after = subs_b == 1
    i_first = jnp.where(is_after, cr_b, cl_b)
    i_second = jnp.where(is_after, cl_b, cr_b)
    pk_f = _loop_gather(vc, i_first, _NB)
    pk_s = _loop_gather(vc, i_second, _NB)
    is_comb = ecat_b == 2
    c_a = jnp.where(is_comb, pk_f >> 3, cpm_b)
    v_a = jnp.where(is_comb, pk_f & 7, vpm_b)
    c_b = jnp.where(is_comb, pk_s >> 3, 0)
    v_b = pk_s & 7

    cnt_ref[0] = (c_a + c_b).astype(jnp.float32)

    e1 = e1_ref[...]
    e2 = e2_ref[...]
    zero = jnp.zeros((1, 1, 1), jnp.float32)
    p3 = lax.broadcasted_iota(jnp.int32, (_MO, 1, 1), 0)
    cab = c_a + c_b
    for k in range(_NB):
        ca3 = lax.broadcast_in_dim(c_a[k:k + 1, :], (1, 1, _B), (1, 2))
        cab3 = lax.broadcast_in_dim(cab[k:k + 1, :], (1, 1, _B), (1, 2))
        va3 = lax.broadcast_in_dim(v_a[k:k + 1, :], (1, 1, _B), (1, 2))
        vb3 = lax.broadcast_in_dim(v_b[k:k + 1, :], (1, 1, _B), (1, 2))
        ea = jnp.where(va3 == 1, e1, jnp.where(va3 == 2, e2, zero))
        eb = jnp.where(vb3 == 1, e1, jnp.where(vb3 == 2, e2, zero))
        in_a = p3 < ca3
        in_ab = p3 < cab3
        out_ref[k] = jnp.where(in_a, ea, jnp.where(in_ab, eb, zero))


def kernel(node_cats, node_subs, node_mask, child_left, child_right, action_embed):
    mask_i = node_mask.astype(jnp.int32)
    ct, st, mt = node_cats.T, node_subs.T, mask_i.T
    lt, rt = child_left.T, child_right.T
    g = _N // _NB
    full_spec = pl.BlockSpec((_N, _B), lambda i: (0, 0))
    blk_spec = pl.BlockSpec((1, _NB, _B), lambda i: (i, 0, 0))
    evec_spec = pl.BlockSpec((1, _D, 1), lambda i: (0, 0, 0))
    out, cnt_t = pl.pallas_call(
        _body,
        grid=(g,),
        in_specs=[full_spec] * 5 + [blk_spec] * 5 + [evec_spec] * 2,
        out_specs=[pl.BlockSpec((_NB, _MO, _D, _B), lambda i: (i, 0, 0, 0)),
                   pl.BlockSpec((1, _NB, _B), lambda i: (i, 0, 0))],
        out_shape=[jax.ShapeDtypeStruct((_N, _MO, _D, _B), jnp.float32),
                   jax.ShapeDtypeStruct((g, _NB, _B), jnp.float32)],
    )(ct, st, mt, lt, rt,
      ct.reshape(g, _NB, _B), st.reshape(g, _NB, _B), mt.reshape(g, _NB, _B),
      lt.reshape(g, _NB, _B), rt.reshape(g, _NB, _B),
      action_embed[1].reshape(1, _D, 1), action_embed[2].reshape(1, _D, 1))

    return jnp.transpose(out, (3, 0, 1, 2)), cnt_t.reshape(_N, _B).T
